# baseline (speedup 1.0000x reference)
import functools

import jax
import jax.numpy as jnp
from jax import lax
from jax.experimental import pallas as pl
from jax.experimental.pallas import tpu as pltpu
from jax.experimental.pallas import tpu_sc as plsc

_NUM_CLASSES = 100000
_HIDDEN = 64
_CTX = 16
_B, _L = 16384, 200
_N = _B * _L

_ROWS_BLK = 2000

_NW = 32
_PER_W = _N // _NW
_C = 1600
_CHUNKS = _PER_W // _C
_SH_BLK = 6256

_MASK_BLK = 8192


def _table_body(emb_ref, w1_ref, b1_ref, w2_ref, b2_ref, out_ref):
    z = jnp.maximum(emb_ref[...], 0.0)
    h = jnp.dot(z, w1_ref[...], preferred_element_type=jnp.float32) + b1_ref[...]
    h = jnp.maximum(h, 0.0)
    out_ref[...] = (
        jnp.dot(h, w2_ref[...], preferred_element_type=jnp.float32) + b2_ref[...]
    ).astype(jnp.bfloat16)


def _build_table(emb, W1, b1, W2, b2):
    grid = _NUM_CLASSES // _ROWS_BLK
    return pl.pallas_call(
        _table_body,
        grid=(grid,),
        in_specs=[
            pl.BlockSpec((_ROWS_BLK, _HIDDEN), lambda i: (i, 0)),
            pl.BlockSpec((_HIDDEN, _HIDDEN), lambda i: (0, 0)),
            pl.BlockSpec((1, _HIDDEN), lambda i: (0, 0)),
            pl.BlockSpec((_HIDDEN, _CTX), lambda i: (0, 0)),
            pl.BlockSpec((1, _CTX), lambda i: (0, 0)),
        ],
        out_specs=pl.BlockSpec((_ROWS_BLK, _CTX), lambda i: (i, 0)),
        out_shape=jax.ShapeDtypeStruct((_NUM_CLASSES, _CTX), jnp.bfloat16),
    )(emb, W1, b1.reshape(1, _HIDDEN), W2, b2.reshape(1, _CTX))


def _sc_body(table_hbm, x_hbm, out_hbm, table_sh, idx_v, rows_v, sem):
    sid = lax.axis_index("s")
    wid = sid * 2 + lax.axis_index("c")
    base = wid * _PER_W

    start = sid * _SH_BLK
    last = _NUM_CLASSES - 15 * _SH_BLK

    @pl.when(sid < 15)
    def _():
        pltpu.sync_copy(
            table_hbm.at[pl.ds(start, _SH_BLK)], table_sh.at[pl.ds(start, _SH_BLK)]
        )

    @pl.when(sid == 15)
    def _():
        pltpu.sync_copy(
            table_hbm.at[pl.ds(15 * _SH_BLK, last)],
            table_sh.at[pl.ds(15 * _SH_BLK, last)],
        )

    plsc.subcore_barrier()

    def chunk(i, carry):
        off = base + i * _C
        pltpu.sync_copy(x_hbm.at[pl.ds(off, _C)], idx_v)
        pltpu.async_copy(table_sh.at[idx_v], rows_v, sem).wait()
        pltpu.sync_copy(rows_v, out_hbm.at[pl.ds(off, _C)])
        return carry

    lax.fori_loop(0, _CHUNKS, chunk, 0)


def _gather(table, xf):
    mesh = plsc.VectorSubcoreMesh(core_axis_name="c", subcore_axis_name="s")
    k = functools.partial(
        pl.kernel,
        mesh=mesh,
        out_type=jax.ShapeDtypeStruct((_N, _CTX), jnp.bfloat16),
        scratch_types=[
            pltpu.VMEM_SHARED((_NUM_CLASSES, _CTX), jnp.bfloat16),
            pltpu.VMEM((_C,), jnp.int32),
            pltpu.VMEM((_C, _CTX), jnp.bfloat16),
            pltpu.SemaphoreType.DMA,
        ],
        compiler_params=pltpu.CompilerParams(use_tc_tiling_on_sc=False),
    )(_sc_body)
    return k(table, xf)


def _mask_body(rows_ref, mask_ref, out_ref):
    out_ref[...] = rows_ref[...].astype(jnp.float32) * mask_ref[...]


def _apply_mask(rows, mf):
    grid = _N // _MASK_BLK
    return pl.pallas_call(
        _mask_body,
        grid=(grid,),
        in_specs=[
            pl.BlockSpec((_MASK_BLK, _CTX), lambda i: (i, 0)),
            pl.BlockSpec((_MASK_BLK, 1), lambda i: (i, 0)),
        ],
        out_specs=pl.BlockSpec((_MASK_BLK, _CTX), lambda i: (i, 0)),
        out_shape=jax.ShapeDtypeStruct((_N, _CTX), jnp.float32),
    )(rows, mf.reshape(_N, 1))


def kernel(x, mask, emb, W1, b1, W2, b2):
    table = _build_table(emb, W1, b1, W2, b2)
    rows = _gather(table, x.reshape(_N))
    out = _apply_mask(rows, mask.reshape(_N))
    return out.reshape(_B, _L, _CTX)


# speedup vs baseline: 1.7329x; 1.7329x over previous
import functools

import jax
import jax.numpy as jnp
from jax import lax
from jax.experimental import pallas as pl
from jax.experimental.pallas import tpu as pltpu
from jax.experimental.pallas import tpu_sc as plsc

_NUM_CLASSES = 100000
_HIDDEN = 64
_CTX = 16
_B, _L = 16384, 200
_N = _B * _L

_ROWS_BLK = 2000

_NW = 32
_PER_W = _N // _NW
_C = 1600
_CHUNKS = _PER_W // _C
_SH_BLK = 6256

_MASK_BLK = 8192


def _table_body(emb_ref, w1_ref, b1_ref, w2_ref, b2_ref, out_ref):
    z = jnp.maximum(emb_ref[...], 0.0)
    h = jnp.dot(z, w1_ref[...], preferred_element_type=jnp.float32) + b1_ref[...]
    h = jnp.maximum(h, 0.0)
    out_ref[...] = (
        jnp.dot(h, w2_ref[...], preferred_element_type=jnp.float32) + b2_ref[...]
    ).astype(jnp.bfloat16)


def _build_table(emb, W1, b1, W2, b2):
    grid = _NUM_CLASSES // _ROWS_BLK
    return pl.pallas_call(
        _table_body,
        grid=(grid,),
        in_specs=[
            pl.BlockSpec((_ROWS_BLK, _HIDDEN), lambda i: (i, 0)),
            pl.BlockSpec((_HIDDEN, _HIDDEN), lambda i: (0, 0)),
            pl.BlockSpec((1, _HIDDEN), lambda i: (0, 0)),
            pl.BlockSpec((_HIDDEN, _CTX), lambda i: (0, 0)),
            pl.BlockSpec((1, _CTX), lambda i: (0, 0)),
        ],
        out_specs=pl.BlockSpec((_ROWS_BLK, _CTX), lambda i: (i, 0)),
        out_shape=jax.ShapeDtypeStruct((_NUM_CLASSES, _CTX), jnp.bfloat16),
    )(emb, W1, b1.reshape(1, _HIDDEN), W2, b2.reshape(1, _CTX))


def _sc_body(table_hbm, x_hbm, out_hbm, table_sh, idx_v, rows_v, sem):
    sid = lax.axis_index("s")
    wid = sid * 2 + lax.axis_index("c")
    base = wid * _PER_W

    start = sid * _SH_BLK
    last = _NUM_CLASSES - 15 * _SH_BLK

    @pl.when(sid < 15)
    def _():
        pltpu.sync_copy(
            table_hbm.at[pl.ds(start, _SH_BLK)], table_sh.at[pl.ds(start, _SH_BLK)]
        )

    @pl.when(sid == 15)
    def _():
        pltpu.sync_copy(
            table_hbm.at[pl.ds(15 * _SH_BLK, last)],
            table_sh.at[pl.ds(15 * _SH_BLK, last)],
        )

    plsc.subcore_barrier()

    def chunk(i, carry):
        off = base + i * _C
        pltpu.sync_copy(x_hbm.at[pl.ds(off, _C)], idx_v)
        pltpu.async_copy(table_sh.at[idx_v], rows_v, sem).wait()
        pltpu.sync_copy(rows_v, out_hbm.at[pl.ds(off, _C)])
        return carry

    lax.fori_loop(0, _CHUNKS, chunk, 0)


def _gather(table, xf):
    mesh = plsc.VectorSubcoreMesh(core_axis_name="c", subcore_axis_name="s")
    k = functools.partial(
        pl.kernel,
        mesh=mesh,
        out_type=jax.ShapeDtypeStruct((_N, _CTX), jnp.bfloat16),
        scratch_types=[
            pltpu.VMEM_SHARED((_NUM_CLASSES, _CTX), jnp.bfloat16),
            pltpu.VMEM((_C,), jnp.int32),
            pltpu.VMEM((_C, _CTX), jnp.bfloat16),
            pltpu.SemaphoreType.DMA,
        ],
        compiler_params=pltpu.CompilerParams(use_tc_tiling_on_sc=False),
    )(_sc_body)
    return k(table, xf)


_B_BLK = 128


def _mask_body(rows_ref, mask_ref, out_ref):
    r = rows_ref[...].astype(jnp.float32)
    v = r * mask_ref[...][:, :, None]
    out_ref[...] = v.reshape(_B_BLK, _L * _CTX).T


def _apply_mask(rows, mask2d):
    grid = _B // _B_BLK
    out2d = pl.pallas_call(
        _mask_body,
        grid=(grid,),
        in_specs=[
            pl.BlockSpec((_B_BLK, _L, _CTX), lambda i: (i, 0, 0)),
            pl.BlockSpec((_B_BLK, _L), lambda i: (i, 0)),
        ],
        out_specs=pl.BlockSpec((_L * _CTX, _B_BLK), lambda i: (0, i)),
        out_shape=jax.ShapeDtypeStruct((_L * _CTX, _B), jnp.float32),
    )(rows.reshape(_B, _L, _CTX), mask2d)
    return out2d.reshape(_L, _CTX, _B).transpose(2, 0, 1)


def kernel(x, mask, emb, W1, b1, W2, b2):
    table = _build_table(emb, W1, b1, W2, b2)
    rows = _gather(table, x.reshape(_N))
    return _apply_mask(rows, mask)
